# baseline (speedup 1.0000x reference)
import functools
import math

import jax
import jax.numpy as jnp
from jax.experimental import pallas as pl
from jax.experimental.pallas import tpu as pltpu

_HIGH = jax.lax.Precision.HIGHEST


def _body(H, kk, scale, x_ref, wr_ref, br_ref, wo_ref, bout_ref, s_ref,
          o_ref, bias_scr, bcs_scr):
    h = pl.program_id(1)

    @pl.when(h == 0)
    def _():
        bias = jnp.log(s_ref[0] + 1e-8)
        bias_scr[...] = bias
        bcs_scr[...] = jnp.sum(bias, axis=0, keepdims=True)

    xb = x_ref[0]
    wq = wr_ref[pl.ds(h, 1)][0]
    wk = wr_ref[pl.ds(H + h, 1)][0]
    wv = wr_ref[pl.ds(2 * H + h, 1)][0]
    bq = br_ref[pl.ds(h, 1)][0]
    bk = br_ref[pl.ds(H + h, 1)][0]
    bv = br_ref[pl.ds(2 * H + h, 1)][0]

    dn_t = (((1,), (1,)), ((), ()))
    q = jax.lax.dot_general(xb, wq, dn_t, precision=jax.lax.Precision.DEFAULT) + bq[None, :]
    k = jax.lax.dot_general(xb, wk, dn_t, precision=jax.lax.Precision.DEFAULT) + bk[None, :]
    v = jax.lax.dot_general(xb, wv, dn_t, precision=jax.lax.Precision.DEFAULT) + bv[None, :]

    L = k.shape[0]
    k_ext = jnp.concatenate(
        [k, jnp.sum(k, axis=0, keepdims=True),
         jnp.zeros((7, k.shape[1]), jnp.float32)], axis=0)
    sc_ext = jax.lax.dot_general(k_ext, q, dn_t,
                                 precision=jax.lax.Precision.DEFAULT) * scale
    scores = sc_ext[:L] + bias_scr[...]
    mu = (sc_ext[L:L + 1] + bcs_scr[...]) * (1.0 / L)

    m = jnp.max(scores, axis=0, keepdims=True)
    rmin = jnp.min(scores, axis=0, keepdims=True)
    spread = m - mu
    lo_c = mu + 0.18 * spread
    hi_c = mu + 0.55 * spread
    f_lo = jnp.sum((scores >= lo_c).astype(jnp.float32), axis=0,
                   keepdims=True)
    f_hi = jnp.sum((scores >= hi_c).astype(jnp.float32), axis=0,
                   keepdims=True)
    hi_eps = m + (jnp.abs(m) * 1e-6 + 1e-30)
    ok_lo = f_lo >= kk
    ok_hi = f_hi < kk
    lo0 = jnp.where(ok_lo, lo_c, rmin)
    hi0 = jnp.where(ok_hi, hi_c, hi_eps)
    c_hi0 = jnp.where(ok_hi, f_hi, 0.0)

    def step(_, state):
        lo, hi, c_hi = state
        mid = 0.5 * (lo + hi)
        cnt = jnp.sum((scores >= mid).astype(jnp.float32), axis=0,
                      keepdims=True)
        ge = cnt >= kk
        return (jnp.where(ge, mid, lo), jnp.where(ge, hi, mid),
                jnp.where(ge, c_hi, cnt))

    lo, hi, c_hi = jax.lax.fori_loop(0, 8, step, (lo0, hi0, c_hi0))

    r = kk - c_hi.astype(jnp.int32)
    done0 = (r <= 0).astype(jnp.int32)
    thr0 = jnp.where(done0 == 1, hi, lo)

    def ext_cond(state):
        done, _, _, _ = state
        return jnp.min(done) == 0

    def ext_body(state):
        done, r, thr, ub = state
        cand = (scores >= lo) & (scores < ub)
        mc = jnp.max(jnp.where(cand, scores, -jnp.inf), axis=0,
                     keepdims=True)
        c_m = jnp.sum((scores == mc).astype(jnp.int32), axis=0,
                      keepdims=True)
        active = done == 0
        take = active & (r <= c_m)
        thr = jnp.where(take, mc, thr)
        done = jnp.where(take, 1, done)
        cont = active & jnp.logical_not(take)
        r = jnp.where(cont, r - c_m, r)
        ub = jnp.where(cont, mc, ub)
        return done, r, thr, ub

    _, _, thr, _ = jax.lax.while_loop(
        ext_cond, ext_body, (done0, r, thr0, hi))

    sel = scores >= thr
    p = jnp.where(sel, jnp.exp(scores - m), 0.0)

    vx = jnp.concatenate([v, jnp.ones((v.shape[0], 1), jnp.float32)],
                         axis=1)
    dn_n = (((1,), (0,)), ((), ()))
    dn_c0 = (((0,), (0,)), ((), ()))
    hd = v.shape[1]
    o_ext = jax.lax.dot_general(p, vx, dn_c0,
                                precision=jax.lax.Precision.DEFAULT)
    o = o_ext[:, :hd] * (1.0 / o_ext[:, hd:hd + 1])
    proj = jax.lax.dot_general(o, wo_ref[pl.ds(h, 1)][0], dn_n,
                               precision=jax.lax.Precision.DEFAULT)

    @pl.when(h == 0)
    def _():
        o_ref[0] = proj + bout_ref[0][None, :]

    @pl.when(h != 0)
    def _():
        o_ref[0] = o_ref[0] + proj


def kernel(x, Wqkv, bqkv, Wout, bout, S_struc):
    L, B, D = x.shape
    H = S_struc.shape[0]
    hd = D // H
    kk = max(1, int(0.1 * L))
    scale = 1.0 / math.sqrt(hd)

    Wr = Wqkv.reshape(3 * H, hd, D)
    br = bqkv.reshape(3 * H, hd)
    Wo = jnp.transpose(Wout.reshape(D, H, hd), (1, 2, 0))
    bo = bout.reshape(1, D)

    body = functools.partial(_body, H, kk, scale)
    xt = jnp.transpose(x, (1, 0, 2))

    out = pl.pallas_call(
        body,
        grid=(B, H),
        in_specs=[
            pl.BlockSpec((1, L, D), lambda b, h: (b, 0, 0)),
            pl.BlockSpec((3 * H, hd, D), lambda b, h: (0, 0, 0)),
            pl.BlockSpec((3 * H, hd), lambda b, h: (0, 0)),
            pl.BlockSpec((H, hd, D), lambda b, h: (0, 0, 0)),
            pl.BlockSpec((1, D), lambda b, h: (0, 0)),
            pl.BlockSpec((1, L, L), lambda b, h: (b, 0, 0)),
        ],
        out_specs=pl.BlockSpec((1, L, D), lambda b, h: (b, 0, 0)),
        out_shape=jax.ShapeDtypeStruct((B, L, D), jnp.float32),
        scratch_shapes=[pltpu.VMEM((L, L), jnp.float32),
                        pltpu.VMEM((1, L), jnp.float32)],
        compiler_params=pltpu.CompilerParams(
            dimension_semantics=("arbitrary", "arbitrary")),
    )(xt, Wr, br, Wo, bo, jnp.transpose(S_struc, (0, 2, 1)))
    return jnp.transpose(out, (1, 0, 2))


# speedup vs baseline: 1.0544x; 1.0544x over previous
import functools
import math

import jax
import jax.numpy as jnp
from jax.experimental import pallas as pl
from jax.experimental.pallas import tpu as pltpu

_HIGH = jax.lax.Precision.HIGHEST


def _body(H, kk, scale, x_ref, wr_ref, br_ref, wo_ref, bout_ref, s_ref,
          o_ref, bias_scr, bcs_scr):
    h = pl.program_id(1)

    @pl.when(h == 0)
    def _():
        bias = jnp.log(s_ref[0] + 1e-8)
        bias_scr[...] = bias
        bcs_scr[...] = jnp.sum(bias, axis=0, keepdims=True)

    xb = x_ref[0]
    wq = wr_ref[pl.ds(h, 1)][0]
    wk = wr_ref[pl.ds(H + h, 1)][0]
    wv = wr_ref[pl.ds(2 * H + h, 1)][0]
    bq = br_ref[pl.ds(h, 1)][0]
    bk = br_ref[pl.ds(H + h, 1)][0]
    bv = br_ref[pl.ds(2 * H + h, 1)][0]

    dn_t = (((1,), (1,)), ((), ()))
    q = jax.lax.dot_general(xb, wq, dn_t, precision=jax.lax.Precision.DEFAULT) + bq[None, :]
    k = jax.lax.dot_general(xb, wk, dn_t, precision=jax.lax.Precision.DEFAULT) + bk[None, :]
    v = jax.lax.dot_general(xb, wv, dn_t, precision=jax.lax.Precision.DEFAULT) + bv[None, :]

    L = k.shape[0]
    k_ext = jnp.concatenate(
        [k, jnp.sum(k, axis=0, keepdims=True),
         jnp.zeros((7, k.shape[1]), jnp.float32)], axis=0)
    sc_ext = jax.lax.dot_general(k_ext, q, dn_t,
                                 precision=jax.lax.Precision.DEFAULT) * scale
    scores = sc_ext[:L] + bias_scr[...]
    mu = (sc_ext[L:L + 1] + bcs_scr[...]) * (1.0 / L)

    m = jnp.max(scores, axis=0, keepdims=True)
    rmin = jnp.min(scores, axis=0, keepdims=True)
    spread = m - mu
    lo_c = mu + 0.18 * spread
    hi_c = mu + 0.55 * spread
    f_lo = jnp.sum((scores >= lo_c).astype(jnp.float32), axis=0,
                   keepdims=True)
    f_hi = jnp.sum((scores >= hi_c).astype(jnp.float32), axis=0,
                   keepdims=True)
    hi_eps = m + (jnp.abs(m) * 1e-6 + 1e-30)
    ok_lo = f_lo >= kk
    ok_hi = f_hi < kk
    lo0 = jnp.where(ok_lo, lo_c, rmin)
    hi0 = jnp.where(ok_hi, hi_c, hi_eps)
    c_hi0 = jnp.where(ok_hi, f_hi, 0.0)

    def step(_, state):
        lo, hi, c_hi = state
        mid = 0.5 * (lo + hi)
        cnt = jnp.sum((scores >= mid).astype(jnp.float32), axis=0,
                      keepdims=True)
        ge = cnt >= kk
        return (jnp.where(ge, mid, lo), jnp.where(ge, hi, mid),
                jnp.where(ge, c_hi, cnt))

    lo, hi, c_hi = jax.lax.fori_loop(0, 10, step, (lo0, hi0, c_hi0))

    r = kk - c_hi.astype(jnp.int32)
    done0 = (r <= 0).astype(jnp.int32)
    thr0 = jnp.where(done0 == 1, hi, lo)

    def ext_cond(state):
        done, _, _, _ = state
        return jnp.min(done) == 0

    def ext_body(state):
        done, r, thr, ub = state
        cand = (scores >= lo) & (scores < ub)
        mc = jnp.max(jnp.where(cand, scores, -jnp.inf), axis=0,
                     keepdims=True)
        c_m = jnp.sum((scores == mc).astype(jnp.int32), axis=0,
                      keepdims=True)
        active = done == 0
        take = active & (r <= c_m)
        thr = jnp.where(take, mc, thr)
        done = jnp.where(take, 1, done)
        cont = active & jnp.logical_not(take)
        r = jnp.where(cont, r - c_m, r)
        ub = jnp.where(cont, mc, ub)
        return done, r, thr, ub

    _, _, thr, _ = jax.lax.while_loop(
        ext_cond, ext_body, (done0, r, thr0, hi))

    sel = scores >= thr
    p = jnp.where(sel, jnp.exp(scores - m), 0.0)

    vx = jnp.concatenate([v, jnp.ones((v.shape[0], 1), jnp.float32)],
                         axis=1)
    dn_n = (((1,), (0,)), ((), ()))
    dn_c0 = (((0,), (0,)), ((), ()))
    hd = v.shape[1]
    o_ext = jax.lax.dot_general(p, vx, dn_c0,
                                precision=jax.lax.Precision.DEFAULT)
    o = o_ext[:, :hd] * (1.0 / o_ext[:, hd:hd + 1])
    proj = jax.lax.dot_general(o, wo_ref[pl.ds(h, 1)][0], dn_n,
                               precision=jax.lax.Precision.DEFAULT)

    @pl.when(h == 0)
    def _():
        o_ref[0] = proj + bout_ref[0][None, :]

    @pl.when(h != 0)
    def _():
        o_ref[0] = o_ref[0] + proj


def kernel(x, Wqkv, bqkv, Wout, bout, S_struc):
    L, B, D = x.shape
    H = S_struc.shape[0]
    hd = D // H
    kk = max(1, int(0.1 * L))
    scale = 1.0 / math.sqrt(hd)

    Wr = Wqkv.reshape(3 * H, hd, D)
    br = bqkv.reshape(3 * H, hd)
    Wo = jnp.transpose(Wout.reshape(D, H, hd), (1, 2, 0))
    bo = bout.reshape(1, D)

    body = functools.partial(_body, H, kk, scale)
    xt = jnp.transpose(x, (1, 0, 2))

    out = pl.pallas_call(
        body,
        grid=(B, H),
        in_specs=[
            pl.BlockSpec((1, L, D), lambda b, h: (b, 0, 0)),
            pl.BlockSpec((3 * H, hd, D), lambda b, h: (0, 0, 0)),
            pl.BlockSpec((3 * H, hd), lambda b, h: (0, 0)),
            pl.BlockSpec((H, hd, D), lambda b, h: (0, 0, 0)),
            pl.BlockSpec((1, D), lambda b, h: (0, 0)),
            pl.BlockSpec((1, L, L), lambda b, h: (b, 0, 0)),
        ],
        out_specs=pl.BlockSpec((1, L, D), lambda b, h: (b, 0, 0)),
        out_shape=jax.ShapeDtypeStruct((B, L, D), jnp.float32),
        scratch_shapes=[pltpu.VMEM((L, L), jnp.float32),
                        pltpu.VMEM((1, L), jnp.float32)],
        compiler_params=pltpu.CompilerParams(
            dimension_semantics=("arbitrary", "arbitrary")),
    )(xt, Wr, br, Wo, bo, jnp.transpose(S_struc, (0, 2, 1)))
    return jnp.transpose(out, (1, 0, 2))
